# Initial kernel scaffold; baseline (speedup 1.0000x reference)
import jax
import jax.numpy as jnp
from jax.experimental import pallas as pl


def kernel(z, idx_i, idx_j, d_ij, phi_ij):
    raise NotImplementedError("write your pallas kernel here")


# speedup vs baseline: 6.6530x; 6.6530x over previous
import functools

import numpy as np
import jax
import jax.numpy as jnp
from jax import lax
from jax.experimental import pallas as pl
from jax.experimental.pallas import tpu as pltpu
from jax.experimental.pallas import tpu_sc as plsc

_ORB = '1s 2s 2p 3s 3p 4s 3d 4p 5s 4d 5p 6s 4f 5d 6p 7s 5f 6d 7p 6f 7d 7f'.split()
_NE = dict(s=2, p=6, d=10, f=14)


def _econf(n):
    cnt, last, cfg = 0, -1, []
    for o in _ORB:
        if cnt < n:
            cfg.append(_NE[o[-1]])
            cnt += _NE[o[-1]]
            last += 1
        else:
            cfg.append(0)
    if cnt > n:
        cfg[last] -= cnt - n
    return cfg


_TABLE = np.array([_econf(i) for i in range(128)], dtype=np.float32)

NC, NS = 2, 16
NW = NC * NS
ORB = 22
D_EDGE = 16
W = 2 * ORB + 1 + D_EDGE

E = 1600000
EPW = E // NW
C = 512
NFULL = EPW // C
TAIL = EPW - NFULL * C

_mesh = plsc.VectorSubcoreMesh(core_axis_name="c", subcore_axis_name="s")
_params = pltpu.CompilerParams(use_tc_tiling_on_sc=False,
                               needs_layout_passes=False)


def _wid():
    return lax.axis_index("s") * NC + lax.axis_index("c")


@functools.partial(
    pl.kernel,
    out_type=jax.ShapeDtypeStruct((E * W,), jnp.float32),
    mesh=_mesh,
    scratch_types=[
        pltpu.VMEM((128 * ORB,), jnp.float32),
        pltpu.VMEM((C,), jnp.int32),
        pltpu.VMEM((C,), jnp.int32),
        pltpu.VMEM((C,), jnp.int32),
        pltpu.VMEM((C,), jnp.int32),
        pltpu.VMEM((C, D_EDGE), jnp.float32),
        pltpu.VMEM((C,), jnp.float32),
        pltpu.VMEM((C * W,), jnp.float32),
        pltpu.SemaphoreType.DMA,
    ],
    compiler_params=_params,
)
def _edge_kernel(table_hbm, z22_hbm, idxi_hbm, idxj_hbm, d_hbm, phi_hbm,
                 out_hbm, table_v, idxi_v, idxj_v, zi_v, zj_v, phi_v, d_v,
                 rowf, sem):
    wbase = _wid() * EPW
    pltpu.sync_copy(table_hbm, table_v)
    lanes = lax.iota(jnp.int32, 16)
    lanes_w = lanes * W

    def do_chunk(base, n):
        pltpu.sync_copy(idxi_hbm.at[pl.ds(base, n)], idxi_v.at[pl.ds(0, n)])
        pltpu.sync_copy(idxj_hbm.at[pl.ds(base, n)], idxj_v.at[pl.ds(0, n)])
        cps = [
            pltpu.async_copy(z22_hbm.at[idxi_v.at[pl.ds(0, n)]],
                             zi_v.at[pl.ds(0, n)], sem),
            pltpu.async_copy(z22_hbm.at[idxj_v.at[pl.ds(0, n)]],
                             zj_v.at[pl.ds(0, n)], sem),
            pltpu.async_copy(phi_hbm.at[pl.ds(base, n), :],
                             phi_v.at[pl.ds(0, n), :], sem),
            pltpu.async_copy(d_hbm.at[pl.ds(base, n)], d_v.at[pl.ds(0, n)],
                             sem),
        ]
        for cp in cps:
            cp.wait()

        def grp(g, carry):
            e16 = g * 16 + lanes
            dst = g * (16 * W) + lanes_w
            zbi = zi_v[pl.ds(g * 16, 16)]
            zbj = zj_v[pl.ds(g * 16, 16)]
            for c in range(ORB):
                v = plsc.load_gather(table_v, [zbi + c])
                plsc.store_scatter(rowf, [dst + c], v)
                v = plsc.load_gather(table_v, [zbj + c])
                plsc.store_scatter(rowf, [dst + (ORB + c)], v)
            cfull = jnp.full((16,), 0, jnp.int32)
            for c in range(D_EDGE):
                v = plsc.load_gather(phi_v, [e16, cfull + c])
                plsc.store_scatter(rowf, [dst + (2 * ORB + c)], v)
            v = d_v[pl.ds(g * 16, 16)]
            plsc.store_scatter(rowf, [dst + (W - 1)], v)
            return carry

        lax.fori_loop(0, n // 16, grp, 0)
        pltpu.sync_copy(rowf.at[pl.ds(0, n * W)],
                        out_hbm.at[pl.ds(base * W, n * W)])

    def body(t, carry):
        do_chunk(wbase + t * C, C)
        return carry

    lax.fori_loop(0, NFULL, body, 0)
    do_chunk(wbase + NFULL * C, TAIL)


def kernel(z, idx_i, idx_j, d_ij, phi_ij):
    table = jnp.asarray(_TABLE.reshape(-1))
    z22 = z.astype(jnp.int32) * ORB
    out_flat = _edge_kernel(table, z22, idx_i.astype(jnp.int32),
                            idx_j.astype(jnp.int32),
                            jnp.squeeze(d_ij, -1), phi_ij)
    return out_flat.reshape(E, W)


# speedup vs baseline: 7.0097x; 1.0536x over previous
import functools

import numpy as np
import jax
import jax.numpy as jnp
from jax import lax
from jax.experimental import pallas as pl
from jax.experimental.pallas import tpu as pltpu
from jax.experimental.pallas import tpu_sc as plsc

_ORB = '1s 2s 2p 3s 3p 4s 3d 4p 5s 4d 5p 6s 4f 5d 6p 7s 5f 6d 7p 6f 7d 7f'.split()
_NE = dict(s=2, p=6, d=10, f=14)


def _econf(n):
    cnt, last, cfg = 0, -1, []
    for o in _ORB:
        if cnt < n:
            cfg.append(_NE[o[-1]])
            cnt += _NE[o[-1]]
            last += 1
        else:
            cfg.append(0)
    if cnt > n:
        cfg[last] -= cnt - n
    return cfg


_TABLE = np.array([_econf(i) for i in range(128)], dtype=np.float32)

NC, NS = 2, 16
NW = NC * NS
ORB = 22
D_EDGE = 16
W = 2 * ORB + 1 + D_EDGE

E = 1600000
EPW = E // NW
C = 512
NFULL = EPW // C
TAIL = EPW - NFULL * C
NPAIR = (NFULL - 1) // 2

_mesh = plsc.VectorSubcoreMesh(core_axis_name="c", subcore_axis_name="s")
_params = pltpu.CompilerParams(use_tc_tiling_on_sc=False,
                               needs_layout_passes=False)


def _wid():
    return lax.axis_index("s") * NC + lax.axis_index("c")


@functools.partial(
    pl.kernel,
    out_type=jax.ShapeDtypeStruct((E * W,), jnp.float32),
    mesh=_mesh,
    scratch_types=[
        pltpu.VMEM((128 * ORB,), jnp.float32),
        pltpu.VMEM((C,), jnp.int32),
        pltpu.VMEM((C,), jnp.int32),
        pltpu.VMEM((C,), jnp.int32),
        pltpu.VMEM((C,), jnp.int32),
        pltpu.VMEM((C,), jnp.int32),
        pltpu.VMEM((C,), jnp.int32),
        pltpu.VMEM((C,), jnp.int32),
        pltpu.VMEM((C,), jnp.int32),
        pltpu.VMEM((C, D_EDGE), jnp.float32),
        pltpu.VMEM((C, D_EDGE), jnp.float32),
        pltpu.VMEM((C,), jnp.float32),
        pltpu.VMEM((C,), jnp.float32),
        pltpu.VMEM((C * W,), jnp.float32),
        pltpu.VMEM((C * W,), jnp.float32),
        pltpu.SemaphoreType.DMA,
        pltpu.SemaphoreType.DMA,
        pltpu.SemaphoreType.DMA,
        pltpu.SemaphoreType.DMA,
        pltpu.SemaphoreType.DMA,
        pltpu.SemaphoreType.DMA,
    ],
    compiler_params=_params,
)
def _edge_kernel(table_hbm, z22_hbm, idxi_hbm, idxj_hbm, d_hbm, phi_hbm,
                 out_hbm, table_v, idxi0, idxi1, idxj0, idxj1, zi0, zi1,
                 zj0, zj1, phi0, phi1, dv0, dv1, rowf0, rowf1,
                 isem0, isem1, bsem0, bsem1, osem0, osem1):
    idxi = (idxi0, idxi1)
    idxj = (idxj0, idxj1)
    zi = (zi0, zi1)
    zj = (zj0, zj1)
    phi = (phi0, phi1)
    dv = (dv0, dv1)
    rowf = (rowf0, rowf1)
    isem = (isem0, isem1)
    bsem = (bsem0, bsem1)
    osem = (osem0, osem1)

    wbase = _wid() * EPW
    pltpu.sync_copy(table_hbm, table_v)
    lanes = lax.iota(jnp.int32, 16)
    lanes_w = lanes * W

    def base_of(t):
        return wbase + t * C

    def issue_idx(t, b):
        base = base_of(t)
        pltpu.async_copy(idxi_hbm.at[pl.ds(base, C)], idxi[b], isem[b])
        pltpu.async_copy(idxj_hbm.at[pl.ds(base, C)], idxj[b], isem[b])

    def drain_idx(t, b):
        base = base_of(t)
        pltpu.make_async_copy(idxi_hbm.at[pl.ds(base, C)], idxi[b],
                              isem[b]).wait()
        pltpu.make_async_copy(idxj_hbm.at[pl.ds(base, C)], idxj[b],
                              isem[b]).wait()

    def issue_b(t, b):
        base = base_of(t)
        return [
            pltpu.async_copy(z22_hbm.at[idxi[b]], zi[b], bsem[b]),
            pltpu.async_copy(z22_hbm.at[idxj[b]], zj[b], bsem[b]),
            pltpu.async_copy(phi_hbm.at[pl.ds(base, C), :], phi[b], bsem[b]),
            pltpu.async_copy(d_hbm.at[pl.ds(base, C)], dv[b], bsem[b]),
        ]

    def issue_out(t, b):
        base = base_of(t)
        pltpu.async_copy(rowf[b], out_hbm.at[pl.ds(base * W, C * W)], osem[b])

    def drain_out(t, b):
        base = base_of(t)
        pltpu.make_async_copy(rowf[b], out_hbm.at[pl.ds(base * W, C * W)],
                              osem[b]).wait()

    def compute(b, ngroups):
        zib, zjb, phib, dvb, rowb = zi[b], zj[b], phi[b], dv[b], rowf[b]

        def grp(g, carry):
            e16 = g * 16 + lanes
            dst = g * (16 * W) + lanes_w
            zbi = zib[pl.ds(g * 16, 16)]
            zbj = zjb[pl.ds(g * 16, 16)]
            for c in range(ORB):
                v = plsc.load_gather(table_v, [zbi + c])
                plsc.store_scatter(rowb, [dst + c], v)
                v = plsc.load_gather(table_v, [zbj + c])
                plsc.store_scatter(rowb, [dst + (ORB + c)], v)
            cfull = jnp.full((16,), 0, jnp.int32)
            for c in range(D_EDGE):
                v = plsc.load_gather(phib, [e16, cfull + c])
                plsc.store_scatter(rowb, [dst + (2 * ORB + c)], v)
            v = dvb[pl.ds(g * 16, 16)]
            plsc.store_scatter(rowb, [dst + (W - 1)], v)
            return carry

        lax.fori_loop(0, ngroups, grp, 0)

    def stage(t, b, nxt, drain_prev):
        drain_idx(t, b)
        cps = issue_b(t, b)
        if nxt is not None:
            issue_idx(nxt, 1 - b)
        for cp in cps:
            cp.wait()
        if drain_prev is not None:
            drain_out(drain_prev, b)
        compute(b, C // 16)
        issue_out(t, b)

    issue_idx(0, 0)
    stage(0, 0, 1, None)
    stage(1, 1, 2, None)

    def body(g, carry):
        t = 2 * g
        stage(t, 0, t + 1, t - 2)
        stage(t + 1, 1, t + 2, t - 1)
        return carry

    lax.fori_loop(1, NPAIR, body, 0)

    stage(NFULL - 1, 0, None, NFULL - 3)
    drain_out(NFULL - 2, 1)

    tbase = wbase + NFULL * C
    pltpu.sync_copy(idxi_hbm.at[pl.ds(tbase, TAIL)],
                    idxi[1].at[pl.ds(0, TAIL)])
    pltpu.sync_copy(idxj_hbm.at[pl.ds(tbase, TAIL)],
                    idxj[1].at[pl.ds(0, TAIL)])
    cps = [
        pltpu.async_copy(z22_hbm.at[idxi[1].at[pl.ds(0, TAIL)]],
                         zi[1].at[pl.ds(0, TAIL)], bsem[1]),
        pltpu.async_copy(z22_hbm.at[idxj[1].at[pl.ds(0, TAIL)]],
                         zj[1].at[pl.ds(0, TAIL)], bsem[1]),
        pltpu.async_copy(phi_hbm.at[pl.ds(tbase, TAIL), :],
                         phi[1].at[pl.ds(0, TAIL), :], bsem[1]),
        pltpu.async_copy(d_hbm.at[pl.ds(tbase, TAIL)],
                         dv[1].at[pl.ds(0, TAIL)], bsem[1]),
    ]
    for cp in cps:
        cp.wait()
    compute(1, TAIL // 16)
    pltpu.sync_copy(rowf[1].at[pl.ds(0, TAIL * W)],
                    out_hbm.at[pl.ds(tbase * W, TAIL * W)])
    drain_out(NFULL - 1, 0)


def kernel(z, idx_i, idx_j, d_ij, phi_ij):
    table = jnp.asarray(_TABLE.reshape(-1))
    z22 = z.astype(jnp.int32) * ORB
    out_flat = _edge_kernel(table, z22, idx_i.astype(jnp.int32),
                            idx_j.astype(jnp.int32),
                            jnp.squeeze(d_ij, -1), phi_ij)
    return out_flat.reshape(E, W)
